# baseline (speedup 1.0000x reference)
import functools

import jax
import jax.numpy as jnp
import numpy as np
from jax import lax
from jax.experimental import pallas as pl
from jax.experimental.pallas import tpu as pltpu
from jax.experimental.pallas import tpu_sc as plsc

N = 10000
D = 128
H = 32
G = 64
E = 320000
FH = 12
TV = 4

NSH = 4
FPT = 4
NFG = H // FPT
EPT = E // NSH
CHUNK = 4000
NCHUNK = EPT // CHUNK
CIT = CHUNK // 16
HP = H // 2
NP = 10240
POOLC = 128
NPOOL = NP // POOLC

_F32 = jnp.float32


def _gelu(x):
    return 0.5 * x * (1.0 + lax.erf(x * (2.0 ** -0.5)))


def _leaky_exp(e):
    return jnp.exp(jnp.maximum(e, 0.2 * e))


def _pack_rows(ht):
    lo = lax.bitcast_convert_type(ht[0:HP].astype(jnp.bfloat16),
                                  jnp.int16).astype(jnp.int32) & 0xFFFF
    hi = lax.bitcast_convert_type(ht[HP:H].astype(jnp.bfloat16),
                                  jnp.int16).astype(jnp.int32) << 16
    return hi | lo


def _tc_pre_body(x_ref, w1t_ref, as_ref, ad_ref, ht_ref, aa_ref, hp_ref):
    ht = lax.dot_general(w1t_ref[...], x_ref[...],
                         (((1,), (1,)), ((), ())),
                         preferred_element_type=_F32)
    ht_ref[...] = ht
    hp_ref[...] = _pack_rows(ht)
    aa_ref[0:1, :] = lax.dot_general(as_ref[...], ht, (((1,), (0,)), ((), ())),
                                     preferred_element_type=_F32)
    aa_ref[1:2, :] = lax.dot_general(ad_ref[...], ht, (((1,), (0,)), ((), ())),
                                     preferred_element_type=_F32)


_tc_pre = pl.pallas_call(
    _tc_pre_body,
    out_shape=[jax.ShapeDtypeStruct((H, N), _F32),
               jax.ShapeDtypeStruct((2, N), _F32),
               jax.ShapeDtypeStruct((HP, N), jnp.int32)],
)


def _sc_edge_body(edge_hbm, hp_hbm, aa_hbm, parts_hbm, denp_hbm,
                  hw0, hw1, a0, a1, a2, a3,
                  als_v, ald_v, den_v, src_a, dst_a, src_b, dst_b,
                  sem_sa, sem_da, sem_sb, sem_db):
    c = lax.axis_index("c")
    s = lax.axis_index("s")
    wid = c * 16 + s
    fg = wid % NFG
    sh = wid // NFG
    accs = (a0, a1, a2, a3)

    zero16 = jnp.zeros((16,), _F32)

    @plsc.parallel_loop(0, N // 16, unroll=4)
    def _(i):
        for f in range(FPT):
            accs[f][pl.ds(i * 16, 16)] = zero16
        den_v[pl.ds(i * 16, 16)] = zero16

    ebase = sh * EPT

    def _start(ci, bs, bd, ss, sd):
        off = ebase + ci * CHUNK
        pltpu.async_copy(edge_hbm.at[pl.ds(off, CHUNK)], bs, ss)
        pltpu.async_copy(edge_hbm.at[pl.ds(E + off, CHUNK)], bd, sd)

    def _wait(bs, bd, ss, sd):
        pltpu.make_async_copy(edge_hbm.at[pl.ds(0, CHUNK)], bs, ss).wait()
        pltpu.make_async_copy(edge_hbm.at[pl.ds(0, CHUNK)], bd, sd).wait()

    def _process(sbuf, dbuf):
        @plsc.parallel_loop(0, CIT, unroll=1)
        def _(i):
            srcv = sbuf[pl.ds(i * 16, 16)]
            dstv = dbuf[pl.ds(i * 16, 16)]
            e = plsc.load_gather(als_v, [srcv]) + plsc.load_gather(ald_v, [dstv])
            w = _leaky_exp(e)
            mhi = jnp.int32(-65536)
            for k, hw in enumerate((hw0, hw1)):
                v = plsc.load_gather(hw, [srcv])
                ve = plsc.bitcast(v << 16, _F32)
                vo = plsc.bitcast(v & mhi, _F32)
                plsc.addupdate_scatter(accs[2 * k], [dstv], ve * w)
                plsc.addupdate_scatter(accs[2 * k + 1], [dstv], vo * w)
            plsc.addupdate_scatter(den_v, [dstv], w)

    _start(0, src_a, dst_a, sem_sa, sem_da)
    _start(1, src_b, dst_b, sem_sb, sem_db)
    pltpu.sync_copy(hp_hbm.at[2 * fg], hw0)
    pltpu.sync_copy(hp_hbm.at[2 * fg + 1], hw1)
    pltpu.sync_copy(aa_hbm.at[0], als_v)
    pltpu.sync_copy(aa_hbm.at[1], ald_v)

    def chunk_body(cj, _):
        ca = 2 * cj
        _wait(src_a, dst_a, sem_sa, sem_da)
        _process(src_a, dst_a)

        @pl.when(ca + 2 < NCHUNK)
        def _():
            _start(ca + 2, src_a, dst_a, sem_sa, sem_da)

        _wait(src_b, dst_b, sem_sb, sem_db)
        _process(src_b, dst_b)

        @pl.when(ca + 3 < NCHUNK)
        def _():
            _start(ca + 3, src_b, dst_b, sem_sb, sem_db)

        return 0

    lax.fori_loop(0, NCHUNK // 2, chunk_body, 0)

    pltpu.sync_copy(a0, parts_hbm.at[sh, 2 * fg])
    pltpu.sync_copy(a1, parts_hbm.at[sh, HP + 2 * fg])
    pltpu.sync_copy(a2, parts_hbm.at[sh, 2 * fg + 1])
    pltpu.sync_copy(a3, parts_hbm.at[sh, HP + 2 * fg + 1])

    @pl.when(fg == 0)
    def _():
        pltpu.sync_copy(den_v, denp_hbm.at[sh])


_sc_edge = pl.kernel(
    _sc_edge_body,
    out_type=[jax.ShapeDtypeStruct((NSH, H, N), _F32),
              jax.ShapeDtypeStruct((NSH, N), _F32)],
    mesh=plsc.VectorSubcoreMesh(core_axis_name="c", subcore_axis_name="s"),
    compiler_params=pltpu.CompilerParams(needs_layout_passes=False),
    scratch_types=(
        [pltpu.VMEM((N,), jnp.int32) for _ in range(2)]
        + [pltpu.VMEM((N,), _F32) for _ in range(FPT)]
        + [
            pltpu.VMEM((N,), _F32),
            pltpu.VMEM((N,), _F32),
            pltpu.VMEM((N,), _F32),
            pltpu.VMEM((CHUNK,), jnp.int32),
            pltpu.VMEM((CHUNK,), jnp.int32),
            pltpu.VMEM((CHUNK,), jnp.int32),
            pltpu.VMEM((CHUNK,), jnp.int32),
            pltpu.SemaphoreType.DMA,
            pltpu.SemaphoreType.DMA,
            pltpu.SemaphoreType.DMA,
            pltpu.SemaphoreType.DMA,
        ]
    ),
)


def _tc_combine_body(parts_ref, denp_ref, ht_ref, aa_ref, b_ref,
                     wnt_ref, asn_ref, adn_ref, htn_ref, aan_ref, hpn_ref):
    es = aa_ref[0:1, :] + aa_ref[1:2, :]
    wself = _leaky_exp(es)
    ht = ht_ref[...]
    num = (parts_ref[0] + parts_ref[1] + parts_ref[2] + parts_ref[3]
           + wself * ht)
    den = (denp_ref[0:1, :] + denp_ref[1:2, :] + denp_ref[2:3, :]
           + denp_ref[3:4, :] + wself)
    o = num / (den + 1e-16) + b_ref[...]
    g = _gelu(o)
    htn = lax.dot_general(wnt_ref[...], g, (((1,), (0,)), ((), ())),
                          preferred_element_type=_F32)
    htn_ref[...] = htn
    hpn_ref[...] = _pack_rows(htn)
    aan_ref[0:1, :] = lax.dot_general(asn_ref[...], htn, (((1,), (0,)), ((), ())),
                                      preferred_element_type=_F32)
    aan_ref[1:2, :] = lax.dot_general(adn_ref[...], htn, (((1,), (0,)), ((), ())),
                                      preferred_element_type=_F32)


_tc_combine = pl.pallas_call(
    _tc_combine_body,
    out_shape=[jax.ShapeDtypeStruct((H, N), _F32),
               jax.ShapeDtypeStruct((2, N), _F32),
               jax.ShapeDtypeStruct((HP, N), jnp.int32)],
)


def _tc_head_body(parts_ref, denp_ref, ht_ref, aa_ref, b_ref, batch_ref,
                  wr_ref, br_ref, wm0_ref, bm0_ref, wm1_ref, bm1_ref,
                  wl_ref, bl_ref, out_ref, hf_ref):
    es = aa_ref[0:1, :] + aa_ref[1:2, :]
    wself = _leaky_exp(es)
    num = (parts_ref[0] + parts_ref[1] + parts_ref[2] + parts_ref[3]
           + wself * ht_ref[...])
    den = (denp_ref[0:1, :] + denp_ref[1:2, :] + denp_ref[2:3, :]
           + denp_ref[3:4, :] + wself)
    hf = num / (den + 1e-16) + b_ref[...]
    hf_ref[...] = jnp.zeros((H, NP), _F32)
    hf_ref[:, 0:N] = hf

    batchp = batch_ref[...]
    gids = lax.broadcasted_iota(jnp.int32, (G, NP), 0)
    onehot = (gids == batchp).astype(_F32)
    cnt = jnp.sum(onehot, axis=1, keepdims=True)
    ssum = lax.dot_general(onehot, hf_ref[...], (((1,), (1,)), ((), ())),
                           preferred_element_type=_F32)
    mean = ssum / jnp.maximum(cnt, 1.0)

    gids3 = lax.broadcasted_iota(jnp.int32, (G, 1, POOLC), 0)
    neg = jnp.float32(-jnp.inf)

    def pbody(ci, acc):
        hsl = hf_ref[:, pl.ds(ci * POOLC, POOLC)]
        bsl = batch_ref[0:1, pl.ds(ci * POOLC, POOLC)]
        m3 = bsl.reshape(1, 1, POOLC) == gids3
        sel = jnp.where(m3, hsl.reshape(1, H, POOLC), neg)
        return jnp.maximum(acc, jnp.max(sel, axis=2))

    xmax = lax.fori_loop(0, NPOOL, pbody, jnp.full((G, H), neg, _F32))
    xmax = jnp.where(xmax == neg, 0.0, xmax)

    z = jnp.concatenate([mean, xmax], axis=1)
    z = lax.dot_general(z, wr_ref[...], (((1,), (0,)), ((), ())),
                        preferred_element_type=_F32) + br_ref[...]
    z = jnp.maximum(lax.dot_general(z, wm0_ref[...], (((1,), (0,)), ((), ())),
                                    preferred_element_type=_F32) + bm0_ref[...], 0.0)
    z = jnp.maximum(lax.dot_general(z, wm1_ref[...], (((1,), (0,)), ((), ())),
                                    preferred_element_type=_F32) + bm1_ref[...], 0.0)
    out_ref[...] = lax.dot_general(z, wl_ref[...], (((1,), (0,)), ((), ())),
                                   preferred_element_type=_F32) + bl_ref[...]


_tc_head = pl.pallas_call(
    _tc_head_body,
    out_shape=jax.ShapeDtypeStruct((G, FH * TV), _F32),
    scratch_shapes=[pltpu.VMEM((H, NP), _F32)],
)


_PERM = np.concatenate([np.arange(0, H, 2), np.arange(1, H, 2)])
_BIGP = np.concatenate([_PERM, H + _PERM])


def kernel(x, edge_index, batch, W1, as1, ad1, b1, W2, as2, ad2, b2,
           W3, as3, ad3, b3, Wr, br, Wm0, bm0, Wm1, bm1, Wl, bl):
    batch2 = jnp.concatenate(
        [batch.reshape(1, N), jnp.full((1, NP - N), G, jnp.int32)], axis=1)

    ht1, aa1, hp1 = _tc_pre(x, W1[:, _PERM].T,
                            as1[_PERM].reshape(1, H), ad1[_PERM].reshape(1, H))
    eflat = edge_index.reshape(2 * E)
    p1, d1 = _sc_edge(eflat, hp1, aa1)
    ht2, aa2, hp2 = _tc_combine(p1, d1, ht1, aa1, b1[_PERM].reshape(H, 1),
                                W2[_PERM][:, _PERM].T,
                                as2[_PERM].reshape(1, H),
                                ad2[_PERM].reshape(1, H))
    p2, d2 = _sc_edge(eflat, hp2, aa2)
    ht3, aa3, hp3 = _tc_combine(p2, d2, ht2, aa2, b2[_PERM].reshape(H, 1),
                                W3[_PERM][:, _PERM].T,
                                as3[_PERM].reshape(1, H),
                                ad3[_PERM].reshape(1, H))
    p3, d3 = _sc_edge(eflat, hp3, aa3)
    out = _tc_head(p3, d3, ht3, aa3, b3[_PERM].reshape(H, 1), batch2,
                   Wr[_BIGP], br.reshape(1, H), Wm0, bm0.reshape(1, H),
                   Wm1, bm1.reshape(1, H), Wl, bl.reshape(1, FH * TV))
    return out.reshape(G, FH, TV)


# speedup vs baseline: 1.0016x; 1.0016x over previous
import functools

import jax
import jax.numpy as jnp
import numpy as np
from jax import lax
from jax.experimental import pallas as pl
from jax.experimental.pallas import tpu as pltpu
from jax.experimental.pallas import tpu_sc as plsc

N = 10000
D = 128
H = 32
G = 64
E = 320000
FH = 12
TV = 4

NSH = 4
FPT = 4
NFG = H // FPT
EPT = E // NSH
CHUNK = 4000
NCHUNK = EPT // CHUNK
CIT = CHUNK // 16
HP = H // 2
NP = 10240
POOLC = 128
NPOOL = NP // POOLC

_F32 = jnp.float32


def _gelu(x):
    return 0.5 * x * (1.0 + lax.erf(x * (2.0 ** -0.5)))


def _leaky_exp(e):
    return jnp.exp(jnp.maximum(e, 0.2 * e))


def _pack_rows(ht):
    lo = lax.bitcast_convert_type(ht[0:HP].astype(jnp.bfloat16),
                                  jnp.int16).astype(jnp.int32) & 0xFFFF
    hi = lax.bitcast_convert_type(ht[HP:H].astype(jnp.bfloat16),
                                  jnp.int16).astype(jnp.int32) << 16
    return hi | lo


def _tc_pre_body(x_ref, w1t_ref, as_ref, ad_ref, ht_ref, aa_ref, hp_ref):
    ht = lax.dot_general(w1t_ref[...], x_ref[...],
                         (((1,), (1,)), ((), ())),
                         preferred_element_type=_F32)
    ht_ref[...] = ht
    hp_ref[...] = _pack_rows(ht)
    aa_ref[0:1, :] = lax.dot_general(as_ref[...], ht, (((1,), (0,)), ((), ())),
                                     preferred_element_type=_F32)
    aa_ref[1:2, :] = lax.dot_general(ad_ref[...], ht, (((1,), (0,)), ((), ())),
                                     preferred_element_type=_F32)


_tc_pre = pl.pallas_call(
    _tc_pre_body,
    out_shape=[jax.ShapeDtypeStruct((H, N), _F32),
               jax.ShapeDtypeStruct((2, N), _F32),
               jax.ShapeDtypeStruct((HP, N), jnp.int32)],
)


def _sc_edge_body(edge_hbm, hp_hbm, aa_hbm, parts_hbm, denp_hbm,
                  hw0, hw1, a0, a1, a2, a3,
                  als_v, ald_v, den_v, src_a, dst_a, src_b, dst_b,
                  sem_sa, sem_da, sem_sb, sem_db):
    c = lax.axis_index("c")
    s = lax.axis_index("s")
    wid = c * 16 + s
    fg = wid % NFG
    sh = wid // NFG
    accs = (a0, a1, a2, a3)

    zero16 = jnp.zeros((16,), _F32)

    @plsc.parallel_loop(0, N // 16, unroll=4)
    def _(i):
        for f in range(FPT):
            accs[f][pl.ds(i * 16, 16)] = zero16
        den_v[pl.ds(i * 16, 16)] = zero16

    ebase = sh * EPT

    def _start(ci, bs, bd, ss, sd):
        off = ebase + ci * CHUNK
        pltpu.async_copy(edge_hbm.at[pl.ds(off, CHUNK)], bs, ss)
        pltpu.async_copy(edge_hbm.at[pl.ds(E + off, CHUNK)], bd, sd)

    def _wait(bs, bd, ss, sd):
        pltpu.make_async_copy(edge_hbm.at[pl.ds(0, CHUNK)], bs, ss).wait()
        pltpu.make_async_copy(edge_hbm.at[pl.ds(0, CHUNK)], bd, sd).wait()

    def _process(sbuf, dbuf):
        @plsc.parallel_loop(0, CIT, unroll=2)
        def _(i):
            srcv = sbuf[pl.ds(i * 16, 16)]
            dstv = dbuf[pl.ds(i * 16, 16)]
            e = plsc.load_gather(als_v, [srcv]) + plsc.load_gather(ald_v, [dstv])
            w = _leaky_exp(e)
            mhi = jnp.int32(-65536)
            for k, hw in enumerate((hw0, hw1)):
                v = plsc.load_gather(hw, [srcv])
                ve = plsc.bitcast(v << 16, _F32)
                vo = plsc.bitcast(v & mhi, _F32)
                plsc.addupdate_scatter(accs[2 * k], [dstv], ve * w)
                plsc.addupdate_scatter(accs[2 * k + 1], [dstv], vo * w)
            plsc.addupdate_scatter(den_v, [dstv], w)

    _start(0, src_a, dst_a, sem_sa, sem_da)
    _start(1, src_b, dst_b, sem_sb, sem_db)
    pltpu.sync_copy(hp_hbm.at[2 * fg], hw0)
    pltpu.sync_copy(hp_hbm.at[2 * fg + 1], hw1)
    pltpu.sync_copy(aa_hbm.at[0], als_v)
    pltpu.sync_copy(aa_hbm.at[1], ald_v)

    def chunk_body(cj, _):
        ca = 2 * cj
        _wait(src_a, dst_a, sem_sa, sem_da)
        _process(src_a, dst_a)

        @pl.when(ca + 2 < NCHUNK)
        def _():
            _start(ca + 2, src_a, dst_a, sem_sa, sem_da)

        _wait(src_b, dst_b, sem_sb, sem_db)
        _process(src_b, dst_b)

        @pl.when(ca + 3 < NCHUNK)
        def _():
            _start(ca + 3, src_b, dst_b, sem_sb, sem_db)

        return 0

    lax.fori_loop(0, NCHUNK // 2, chunk_body, 0)

    pltpu.sync_copy(a0, parts_hbm.at[sh, 2 * fg])
    pltpu.sync_copy(a1, parts_hbm.at[sh, HP + 2 * fg])
    pltpu.sync_copy(a2, parts_hbm.at[sh, 2 * fg + 1])
    pltpu.sync_copy(a3, parts_hbm.at[sh, HP + 2 * fg + 1])

    @pl.when(fg == 0)
    def _():
        pltpu.sync_copy(den_v, denp_hbm.at[sh])


_sc_edge = pl.kernel(
    _sc_edge_body,
    out_type=[jax.ShapeDtypeStruct((NSH, H, N), _F32),
              jax.ShapeDtypeStruct((NSH, N), _F32)],
    mesh=plsc.VectorSubcoreMesh(core_axis_name="c", subcore_axis_name="s"),
    compiler_params=pltpu.CompilerParams(needs_layout_passes=False),
    scratch_types=(
        [pltpu.VMEM((N,), jnp.int32) for _ in range(2)]
        + [pltpu.VMEM((N,), _F32) for _ in range(FPT)]
        + [
            pltpu.VMEM((N,), _F32),
            pltpu.VMEM((N,), _F32),
            pltpu.VMEM((N,), _F32),
            pltpu.VMEM((CHUNK,), jnp.int32),
            pltpu.VMEM((CHUNK,), jnp.int32),
            pltpu.VMEM((CHUNK,), jnp.int32),
            pltpu.VMEM((CHUNK,), jnp.int32),
            pltpu.SemaphoreType.DMA,
            pltpu.SemaphoreType.DMA,
            pltpu.SemaphoreType.DMA,
            pltpu.SemaphoreType.DMA,
        ]
    ),
)


def _tc_combine_body(parts_ref, denp_ref, ht_ref, aa_ref, b_ref,
                     wnt_ref, asn_ref, adn_ref, htn_ref, aan_ref, hpn_ref):
    es = aa_ref[0:1, :] + aa_ref[1:2, :]
    wself = _leaky_exp(es)
    ht = ht_ref[...]
    num = (parts_ref[0] + parts_ref[1] + parts_ref[2] + parts_ref[3]
           + wself * ht)
    den = (denp_ref[0:1, :] + denp_ref[1:2, :] + denp_ref[2:3, :]
           + denp_ref[3:4, :] + wself)
    o = num / (den + 1e-16) + b_ref[...]
    g = _gelu(o)
    htn = lax.dot_general(wnt_ref[...], g, (((1,), (0,)), ((), ())),
                          preferred_element_type=_F32)
    htn_ref[...] = htn
    hpn_ref[...] = _pack_rows(htn)
    aan_ref[0:1, :] = lax.dot_general(asn_ref[...], htn, (((1,), (0,)), ((), ())),
                                      preferred_element_type=_F32)
    aan_ref[1:2, :] = lax.dot_general(adn_ref[...], htn, (((1,), (0,)), ((), ())),
                                      preferred_element_type=_F32)


_tc_combine = pl.pallas_call(
    _tc_combine_body,
    out_shape=[jax.ShapeDtypeStruct((H, N), _F32),
               jax.ShapeDtypeStruct((2, N), _F32),
               jax.ShapeDtypeStruct((HP, N), jnp.int32)],
)


def _tc_head_body(parts_ref, denp_ref, ht_ref, aa_ref, b_ref, batch_ref,
                  wr_ref, br_ref, wm0_ref, bm0_ref, wm1_ref, bm1_ref,
                  wl_ref, bl_ref, out_ref, hf_ref):
    es = aa_ref[0:1, :] + aa_ref[1:2, :]
    wself = _leaky_exp(es)
    num = (parts_ref[0] + parts_ref[1] + parts_ref[2] + parts_ref[3]
           + wself * ht_ref[...])
    den = (denp_ref[0:1, :] + denp_ref[1:2, :] + denp_ref[2:3, :]
           + denp_ref[3:4, :] + wself)
    hf = num / (den + 1e-16) + b_ref[...]
    hf_ref[...] = jnp.zeros((H, NP), _F32)
    hf_ref[:, 0:N] = hf

    batchp = batch_ref[...]
    gids = lax.broadcasted_iota(jnp.int32, (G, NP), 0)
    onehot = (gids == batchp).astype(_F32)
    cnt = jnp.sum(onehot, axis=1, keepdims=True)
    ssum = lax.dot_general(onehot, hf_ref[...], (((1,), (1,)), ((), ())),
                           preferred_element_type=_F32)
    mean = ssum / jnp.maximum(cnt, 1.0)

    gids3 = lax.broadcasted_iota(jnp.int32, (G, 1, POOLC), 0)
    neg = jnp.float32(-jnp.inf)

    def pbody(ci, acc):
        hsl = hf_ref[:, pl.ds(ci * POOLC, POOLC)]
        bsl = batch_ref[0:1, pl.ds(ci * POOLC, POOLC)]
        m3 = bsl.reshape(1, 1, POOLC) == gids3
        sel = jnp.where(m3, hsl.reshape(1, H, POOLC), neg)
        return jnp.maximum(acc, jnp.max(sel, axis=2))

    xmax = lax.fori_loop(0, NPOOL, pbody, jnp.full((G, H), neg, _F32))
    xmax = jnp.where(xmax == neg, 0.0, xmax)

    z = jnp.concatenate([mean, xmax], axis=1)
    z = lax.dot_general(z, wr_ref[...], (((1,), (0,)), ((), ())),
                        preferred_element_type=_F32) + br_ref[...]
    z = jnp.maximum(lax.dot_general(z, wm0_ref[...], (((1,), (0,)), ((), ())),
                                    preferred_element_type=_F32) + bm0_ref[...], 0.0)
    z = jnp.maximum(lax.dot_general(z, wm1_ref[...], (((1,), (0,)), ((), ())),
                                    preferred_element_type=_F32) + bm1_ref[...], 0.0)
    out_ref[...] = lax.dot_general(z, wl_ref[...], (((1,), (0,)), ((), ())),
                                   preferred_element_type=_F32) + bl_ref[...]


_tc_head = pl.pallas_call(
    _tc_head_body,
    out_shape=jax.ShapeDtypeStruct((G, FH * TV), _F32),
    scratch_shapes=[pltpu.VMEM((H, NP), _F32)],
)


_PERM = np.concatenate([np.arange(0, H, 2), np.arange(1, H, 2)])
_BIGP = np.concatenate([_PERM, H + _PERM])


def kernel(x, edge_index, batch, W1, as1, ad1, b1, W2, as2, ad2, b2,
           W3, as3, ad3, b3, Wr, br, Wm0, bm0, Wm1, bm1, Wl, bl):
    batch2 = jnp.concatenate(
        [batch.reshape(1, N), jnp.full((1, NP - N), G, jnp.int32)], axis=1)

    ht1, aa1, hp1 = _tc_pre(x, W1[:, _PERM].T,
                            as1[_PERM].reshape(1, H), ad1[_PERM].reshape(1, H))
    eflat = edge_index.reshape(2 * E)
    p1, d1 = _sc_edge(eflat, hp1, aa1)
    ht2, aa2, hp2 = _tc_combine(p1, d1, ht1, aa1, b1[_PERM].reshape(H, 1),
                                W2[_PERM][:, _PERM].T,
                                as2[_PERM].reshape(1, H),
                                ad2[_PERM].reshape(1, H))
    p2, d2 = _sc_edge(eflat, hp2, aa2)
    ht3, aa3, hp3 = _tc_combine(p2, d2, ht2, aa2, b2[_PERM].reshape(H, 1),
                                W3[_PERM][:, _PERM].T,
                                as3[_PERM].reshape(1, H),
                                ad3[_PERM].reshape(1, H))
    p3, d3 = _sc_edge(eflat, hp3, aa3)
    out = _tc_head(p3, d3, ht3, aa3, b3[_PERM].reshape(H, 1), batch2,
                   Wr[_BIGP], br.reshape(1, H), Wm0, bm0.reshape(1, H),
                   Wm1, bm1.reshape(1, H), Wl, bl.reshape(1, FH * TV))
    return out.reshape(G, FH, TV)


# speedup vs baseline: 1.0027x; 1.0010x over previous
import jax
import jax.numpy as jnp
import numpy as np
from jax import lax
from jax.experimental import pallas as pl
from jax.experimental.pallas import tpu as pltpu
from jax.experimental.pallas import tpu_sc as plsc

N = 10000
D = 128
H = 32
G = 64
E = 320000
FH = 12
TV = 4

NSH = 4
FPT = 4
NFG = H // FPT
EPT = E // NSH
CHUNK = 4000
NCHUNK = EPT // CHUNK
CIT = CHUNK // 16
HP = H // 2
NP = 10240
POOLC = 128
NPOOL = NP // POOLC

_F32 = jnp.float32


def _gelu(x):
    return 0.5 * x * (1.0 + lax.erf(x * (2.0 ** -0.5)))


def _leaky_exp(e):
    return jnp.exp(jnp.maximum(e, 0.2 * e))


def _pack_rows(ht):
    lo = lax.bitcast_convert_type(ht[0:HP].astype(jnp.bfloat16),
                                  jnp.int16).astype(jnp.int32) & 0xFFFF
    hi = lax.bitcast_convert_type(ht[HP:H].astype(jnp.bfloat16),
                                  jnp.int16).astype(jnp.int32) << 16
    return hi | lo


def _tc_pre_body(x_ref, w1t_ref, as_ref, ad_ref, ht_ref, aa_ref, hp_ref):
    ht = lax.dot_general(w1t_ref[...], x_ref[...],
                         (((1,), (1,)), ((), ())),
                         preferred_element_type=_F32)
    ht_ref[...] = ht
    hp_ref[...] = _pack_rows(ht)
    aa_ref[0:1, :] = lax.dot_general(as_ref[...], ht, (((1,), (0,)), ((), ())),
                                     preferred_element_type=_F32)
    aa_ref[1:2, :] = lax.dot_general(ad_ref[...], ht, (((1,), (0,)), ((), ())),
                                     preferred_element_type=_F32)


_tc_pre = pl.pallas_call(
    _tc_pre_body,
    out_shape=[jax.ShapeDtypeStruct((H, N), _F32),
               jax.ShapeDtypeStruct((2, N), _F32),
               jax.ShapeDtypeStruct((HP, N), jnp.int32)],
)


def _sc_edge_body(edge_hbm, hp_hbm, aa_hbm, parts_hbm, denp_hbm,
                  hw0, hw1, a0, a1, a2, a3,
                  als_v, ald_v, den_v, src_a, dst_a, src_b, dst_b,
                  sem_sa, sem_da, sem_sb, sem_db):
    c = lax.axis_index("c")
    s = lax.axis_index("s")
    wid = c * 16 + s
    fg = wid % NFG
    sh = wid // NFG
    accs = (a0, a1, a2, a3)

    zero16 = jnp.zeros((16,), _F32)

    @plsc.parallel_loop(0, N // 16, unroll=4)
    def _(i):
        for f in range(FPT):
            accs[f][pl.ds(i * 16, 16)] = zero16
        den_v[pl.ds(i * 16, 16)] = zero16

    ebase = sh * EPT

    def _start(ci, bs, bd, ss, sd):
        off = ebase + ci * CHUNK
        pltpu.async_copy(edge_hbm.at[pl.ds(off, CHUNK)], bs, ss)
        pltpu.async_copy(edge_hbm.at[pl.ds(E + off, CHUNK)], bd, sd)

    def _wait(bs, bd, ss, sd):
        pltpu.make_async_copy(edge_hbm.at[pl.ds(0, CHUNK)], bs, ss).wait()
        pltpu.make_async_copy(edge_hbm.at[pl.ds(0, CHUNK)], bd, sd).wait()

    def _process(sbuf, dbuf):
        @plsc.parallel_loop(0, CIT, unroll=2)
        def _(i):
            srcv = sbuf[pl.ds(i * 16, 16)]
            dstv = dbuf[pl.ds(i * 16, 16)]
            e = plsc.load_gather(als_v, [srcv]) + plsc.load_gather(ald_v, [dstv])
            w = _leaky_exp(e)
            mhi = jnp.int32(-65536)
            for k, hw in enumerate((hw0, hw1)):
                v = plsc.load_gather(hw, [srcv])
                ve = plsc.bitcast(v << 16, _F32)
                vo = plsc.bitcast(v & mhi, _F32)
                plsc.addupdate_scatter(accs[2 * k], [dstv], ve * w)
                plsc.addupdate_scatter(accs[2 * k + 1], [dstv], vo * w)
            plsc.addupdate_scatter(den_v, [dstv], w)

    _start(0, src_a, dst_a, sem_sa, sem_da)
    _start(1, src_b, dst_b, sem_sb, sem_db)
    pltpu.sync_copy(hp_hbm.at[2 * fg], hw0)
    pltpu.sync_copy(hp_hbm.at[2 * fg + 1], hw1)
    pltpu.sync_copy(aa_hbm.at[0], als_v)
    pltpu.sync_copy(aa_hbm.at[1], ald_v)

    def chunk_body(cj, _):
        ca = 2 * cj
        _wait(src_a, dst_a, sem_sa, sem_da)
        _process(src_a, dst_a)

        @pl.when(ca + 2 < NCHUNK)
        def _():
            _start(ca + 2, src_a, dst_a, sem_sa, sem_da)

        _wait(src_b, dst_b, sem_sb, sem_db)
        _process(src_b, dst_b)

        @pl.when(ca + 3 < NCHUNK)
        def _():
            _start(ca + 3, src_b, dst_b, sem_sb, sem_db)

        return 0

    lax.fori_loop(0, NCHUNK // 2, chunk_body, 0)

    pltpu.sync_copy(a0, parts_hbm.at[sh, 2 * fg])
    pltpu.sync_copy(a1, parts_hbm.at[sh, HP + 2 * fg])
    pltpu.sync_copy(a2, parts_hbm.at[sh, 2 * fg + 1])
    pltpu.sync_copy(a3, parts_hbm.at[sh, HP + 2 * fg + 1])

    @pl.when(fg == 0)
    def _():
        pltpu.sync_copy(den_v, denp_hbm.at[sh])


_sc_edge = pl.kernel(
    _sc_edge_body,
    out_type=[jax.ShapeDtypeStruct((NSH, H, N), _F32),
              jax.ShapeDtypeStruct((NSH, N), _F32)],
    mesh=plsc.VectorSubcoreMesh(core_axis_name="c", subcore_axis_name="s"),
    compiler_params=pltpu.CompilerParams(needs_layout_passes=False),
    scratch_types=(
        [pltpu.VMEM((N,), jnp.int32) for _ in range(2)]
        + [pltpu.VMEM((N,), _F32) for _ in range(FPT)]
        + [
            pltpu.VMEM((N,), _F32),
            pltpu.VMEM((N,), _F32),
            pltpu.VMEM((N,), _F32),
            pltpu.VMEM((CHUNK,), jnp.int32),
            pltpu.VMEM((CHUNK,), jnp.int32),
            pltpu.VMEM((CHUNK,), jnp.int32),
            pltpu.VMEM((CHUNK,), jnp.int32),
            pltpu.SemaphoreType.DMA,
            pltpu.SemaphoreType.DMA,
            pltpu.SemaphoreType.DMA,
            pltpu.SemaphoreType.DMA,
        ]
    ),
)


def _tc_combine_body(parts_ref, denp_ref, ht_ref, aa_ref, b_ref,
                     wnt_ref, asn_ref, adn_ref, htn_ref, aan_ref, hpn_ref):
    es = aa_ref[0:1, :] + aa_ref[1:2, :]
    wself = _leaky_exp(es)
    ht = ht_ref[...]
    num = (parts_ref[0] + parts_ref[1] + parts_ref[2] + parts_ref[3]
           + wself * ht)
    den = (denp_ref[0:1, :] + denp_ref[1:2, :] + denp_ref[2:3, :]
           + denp_ref[3:4, :] + wself)
    o = num / (den + 1e-16) + b_ref[...]
    g = _gelu(o)
    htn = lax.dot_general(wnt_ref[...], g, (((1,), (0,)), ((), ())),
                          preferred_element_type=_F32)
    htn_ref[...] = htn
    hpn_ref[...] = _pack_rows(htn)
    aan_ref[0:1, :] = lax.dot_general(asn_ref[...], htn, (((1,), (0,)), ((), ())),
                                      preferred_element_type=_F32)
    aan_ref[1:2, :] = lax.dot_general(adn_ref[...], htn, (((1,), (0,)), ((), ())),
                                      preferred_element_type=_F32)


_tc_combine = pl.pallas_call(
    _tc_combine_body,
    out_shape=[jax.ShapeDtypeStruct((H, N), _F32),
               jax.ShapeDtypeStruct((2, N), _F32),
               jax.ShapeDtypeStruct((HP, N), jnp.int32)],
)


def _tc_head_body(parts_ref, denp_ref, ht_ref, aa_ref, b_ref, batch_ref,
                  wr_ref, br_ref, wm0_ref, bm0_ref, wm1_ref, bm1_ref,
                  wl_ref, bl_ref, out_ref, hf_ref):
    es = aa_ref[0:1, :] + aa_ref[1:2, :]
    wself = _leaky_exp(es)
    num = (parts_ref[0] + parts_ref[1] + parts_ref[2] + parts_ref[3]
           + wself * ht_ref[...])
    den = (denp_ref[0:1, :] + denp_ref[1:2, :] + denp_ref[2:3, :]
           + denp_ref[3:4, :] + wself)
    hf = num / (den + 1e-16) + b_ref[...]
    hf_ref[...] = jnp.zeros((H, NP), _F32)
    hf_ref[:, 0:N] = hf

    batchp = batch_ref[...]
    gids = lax.broadcasted_iota(jnp.int32, (G, NP), 0)
    onehot = (gids == batchp).astype(_F32)
    cnt = jnp.sum(onehot, axis=1, keepdims=True)
    ssum = lax.dot_general(onehot, hf_ref[...], (((1,), (1,)), ((), ())),
                           preferred_element_type=_F32)
    mean = ssum / jnp.maximum(cnt, 1.0)

    gids3 = lax.broadcasted_iota(jnp.int32, (G, 1, POOLC), 0)
    neg = jnp.float32(-jnp.inf)

    def pbody(ci, acc):
        hsl = hf_ref[:, pl.ds(ci * POOLC, POOLC)]
        bsl = batch_ref[0:1, pl.ds(ci * POOLC, POOLC)]
        m3 = bsl.reshape(1, 1, POOLC) == gids3
        sel = jnp.where(m3, hsl.reshape(1, H, POOLC), neg)
        return jnp.maximum(acc, jnp.max(sel, axis=2))

    xmax = lax.fori_loop(0, NPOOL, pbody, jnp.full((G, H), neg, _F32))
    xmax = jnp.where(xmax == neg, 0.0, xmax)

    z = jnp.concatenate([mean, xmax], axis=1)
    z = lax.dot_general(z, wr_ref[...], (((1,), (0,)), ((), ())),
                        preferred_element_type=_F32) + br_ref[...]
    z = jnp.maximum(lax.dot_general(z, wm0_ref[...], (((1,), (0,)), ((), ())),
                                    preferred_element_type=_F32) + bm0_ref[...], 0.0)
    z = jnp.maximum(lax.dot_general(z, wm1_ref[...], (((1,), (0,)), ((), ())),
                                    preferred_element_type=_F32) + bm1_ref[...], 0.0)
    out_ref[...] = lax.dot_general(z, wl_ref[...], (((1,), (0,)), ((), ())),
                                   preferred_element_type=_F32) + bl_ref[...]


_tc_head = pl.pallas_call(
    _tc_head_body,
    out_shape=jax.ShapeDtypeStruct((G, FH * TV), _F32),
    scratch_shapes=[pltpu.VMEM((H, NP), _F32)],
)


_PERM = np.concatenate([np.arange(0, H, 2), np.arange(1, H, 2)])
_BIGP = np.concatenate([_PERM, H + _PERM])


def kernel(x, edge_index, batch, W1, as1, ad1, b1, W2, as2, ad2, b2,
           W3, as3, ad3, b3, Wr, br, Wm0, bm0, Wm1, bm1, Wl, bl):
    batch2 = jnp.concatenate(
        [batch.reshape(1, N), jnp.full((1, NP - N), G, jnp.int32)], axis=1)

    ht1, aa1, hp1 = _tc_pre(x, W1[:, _PERM].T,
                            as1[_PERM].reshape(1, H), ad1[_PERM].reshape(1, H))
    eflat = edge_index.reshape(2 * E)
    p1, d1 = _sc_edge(eflat, hp1, aa1)
    ht2, aa2, hp2 = _tc_combine(p1, d1, ht1, aa1, b1[_PERM].reshape(H, 1),
                                W2[_PERM][:, _PERM].T,
                                as2[_PERM].reshape(1, H),
                                ad2[_PERM].reshape(1, H))
    p2, d2 = _sc_edge(eflat, hp2, aa2)
    ht3, aa3, hp3 = _tc_combine(p2, d2, ht2, aa2, b2[_PERM].reshape(H, 1),
                                W3[_PERM][:, _PERM].T,
                                as3[_PERM].reshape(1, H),
                                ad3[_PERM].reshape(1, H))
    p3, d3 = _sc_edge(eflat, hp3, aa3)
    out = _tc_head(p3, d3, ht3, aa3, b3[_PERM].reshape(H, 1), batch2,
                   Wr[_BIGP], br.reshape(1, H), Wm0, bm0.reshape(1, H),
                   Wm1, bm1.reshape(1, H), Wl, bl.reshape(1, FH * TV))
    return out.reshape(G, FH, TV)
